# baseline (speedup 1.0000x reference)
import functools

import jax
import jax.numpy as jnp
from jax import lax
from jax.experimental import pallas as pl
from jax.experimental.pallas import tpu as pltpu
from jax.experimental.pallas import tpu_sc as plsc

N_NODES = 10000
N_EDGES = 320000
N_REL = 8
HID = 128
HALF = HID // 2
N_GROUPS = 3
G_PAD = 8
LAMDA = 0.01
EPS = 1e-5

NC = 2
NS = 16
NW = NC * NS

NR = N_NODES * N_REL
NR_PAD = 80384
SEG_W = NR_PAD // NW
SEG_T = NR_PAD // NS

CNT_E_TILE = N_EDGES // NS
CNT_CHUNK = 80
CNT_ITERS = CNT_E_TILE // CNT_CHUNK

CHUNK = 128
E_ITERS = 157
E_TILE_PAD = CHUNK * E_ITERS
E_PAD_TOT = NS * E_TILE_PAD
TOT_CHUNKS = E_PAD_TOT // CHUNK
PAD_SEG = NR_PAD - 1

ROW_STRIDE = 624
ROWS_T = 640

_sc_mesh = plsc.VectorSubcoreMesh(
    core_axis_name="c", subcore_axis_name="s", num_cores=NC, num_subcores=NS)
_sc_params = pltpu.CompilerParams(
    needs_layout_passes=False, use_tc_tiling_on_sc=False)


@functools.partial(
    pl.kernel,
    out_type=jax.ShapeDtypeStruct((NR_PAD,), jnp.float32),
    mesh=_sc_mesh,
    scratch_types=[
        pltpu.VMEM_SHARED((NR_PAD,), jnp.float32),
        pltpu.VMEM((3, CHUNK), jnp.int32),
        pltpu.VMEM((3, CHUNK), jnp.int32),
        pltpu.VMEM((3, CHUNK), jnp.int32),
        pltpu.VMEM((CHUNK,), jnp.float32),
        pltpu.VMEM((CHUNK,), jnp.float32),
        pltpu.VMEM((SEG_W,), jnp.float32),
        pltpu.SemaphoreType.DMA,
        pltpu.SemaphoreType.DMA,
        pltpu.SemaphoreType.DMA,
        pltpu.SemaphoreType.DMA,
        pltpu.SemaphoreType.DMA,
        pltpu.SemaphoreType.DMA,
    ],
    compiler_params=_sc_params,
)
def _count_inv(meta_hbm, inv_hbm, acc, m_a, m_b, m_c, ones_v, zeros_v, val_v,
               sem_la, sem_lb, sem_lc, sem_sa, sem_sb, sem_sc2):
    s = lax.axis_index("s")
    c = lax.axis_index("c")
    w = c * NS + s
    zero16 = jnp.zeros((16,), jnp.float32)
    one16 = jnp.ones((16,), jnp.float32)

    @pl.loop(0, CHUNK // 16)
    def _(i):
        ones_v[pl.ds(i * 16, 16)] = one16
        zeros_v[pl.ds(i * 16, 16)] = zero16

    @pl.loop(0, SEG_W // 16)
    def _(i):
        val_v[pl.ds(i * 16, 16)] = zero16

    pltpu.sync_copy(val_v, acc.at[pl.ds(s * SEG_T, SEG_W)])
    pltpu.sync_copy(val_v, acc.at[pl.ds(s * SEG_T + SEG_W, SEG_W)])
    plsc.subcore_barrier()

    c0 = s * E_ITERS

    def start_load(ci, m_v, sem_l):
        pltpu.async_copy(meta_hbm.at[jnp.minimum(ci, TOT_CHUNKS - 1)], m_v,
                         sem_l)

    def wait_load(ci, m_v, sem_l):
        pltpu.make_async_copy(meta_hbm.at[jnp.minimum(ci, TOT_CHUNKS - 1)],
                              m_v, sem_l).wait()

    def cstep(ci2, m_cur, sem_lc, sem_sc, m_nxt2, sem_ln, sem_sn):
        wait_load(0, m_cur, sem_lc)
        pltpu.async_copy(ones_v, acc.at[m_cur.at[1]], sem_sc, add=True)
        pltpu.make_async_copy(zeros_v, acc.at[m_nxt2.at[1]], sem_sn).wait()
        start_load(ci2, m_nxt2, sem_ln)

    pltpu.sync_copy(meta_hbm.at[c0], m_c)
    start_load(c0, m_a, sem_la)
    start_load(c0 + 1, m_b, sem_lb)
    pltpu.async_copy(zeros_v, acc.at[m_c.at[1]], sem_sc2, add=True)

    @pl.loop(0, (E_ITERS - 1) // 3)
    def _(j):
        i = c0 + 3 * j
        cstep(i + 2, m_a, sem_la, sem_sa, m_c, sem_lc, sem_sc2)
        cstep(i + 3, m_b, sem_lb, sem_sb, m_a, sem_la, sem_sa)
        cstep(i + 4, m_c, sem_lc, sem_sc2, m_b, sem_lb, sem_sb)

    cstep(c0 + E_ITERS, m_a, sem_la, sem_sa, m_c, sem_lc, sem_sc2)

    pltpu.make_async_copy(meta_hbm.at[0], m_b, sem_lb).wait()
    pltpu.make_async_copy(meta_hbm.at[0], m_c, sem_lc).wait()
    pltpu.make_async_copy(zeros_v, acc.at[m_a.at[1]], sem_sa).wait()

    plsc.subcore_barrier()
    pltpu.sync_copy(acc.at[pl.ds(w * SEG_W, SEG_W)], val_v)

    @pl.loop(0, SEG_W // 16)
    def _(i):
        v = val_v[pl.ds(i * 16, 16)]
        val_v[pl.ds(i * 16, 16)] = jnp.minimum(v, 1.0) / jnp.maximum(v, 1.0)

    @pl.when(w == NW - 1)
    def _():
        v = val_v[pl.ds(SEG_W - 16, 16)]
        val_v[pl.ds(SEG_W - 16, 16)] = jnp.where(
            lax.iota(jnp.int32, 16) < 15, v, 0.0)

    pltpu.sync_copy(val_v, inv_hbm.at[pl.ds(w * SEG_W, SEG_W)])


@functools.partial(
    pl.kernel,
    out_type=jax.ShapeDtypeStruct((NC * N_NODES, HALF), jnp.float32),
    mesh=_sc_mesh,
    scratch_types=[
        pltpu.VMEM_SHARED((N_NODES, HALF), jnp.float32),
        pltpu.VMEM((CHUNK, HALF), jnp.float32),
        pltpu.VMEM((CHUNK, HALF), jnp.float32),
        pltpu.VMEM((CHUNK, HALF), jnp.float32),
        pltpu.VMEM((3, CHUNK), jnp.int32),
        pltpu.VMEM((3, CHUNK), jnp.int32),
        pltpu.VMEM((3, CHUNK), jnp.int32),
        pltpu.VMEM((CHUNK,), jnp.float32),
        pltpu.VMEM((CHUNK,), jnp.float32),
        pltpu.VMEM((CHUNK,), jnp.float32),
        pltpu.VMEM((CHUNK,), jnp.int32),
        pltpu.VMEM((CHUNK,), jnp.int32),
        pltpu.VMEM((CHUNK,), jnp.int32),
        pltpu.SemaphoreType.DMA,
        pltpu.SemaphoreType.DMA,
        pltpu.SemaphoreType.DMA,
        pltpu.SemaphoreType.DMA,
        pltpu.SemaphoreType.DMA,
        pltpu.SemaphoreType.DMA,
        pltpu.SemaphoreType.DMA,
        pltpu.SemaphoreType.DMA,
        pltpu.SemaphoreType.DMA,
        pltpu.SemaphoreType.DMA,
        pltpu.SemaphoreType.DMA,
        pltpu.SemaphoreType.DMA,
    ],
    compiler_params=_sc_params,
)
def _edge_pass(tbl_lo_hbm, tbl_hi_hbm, meta_hbm, inv_hbm, init_lo_hbm,
               init_hi_hbm, out_hbm, acc, rows_a, rows_b, rows_c, m_a, m_b,
               m_c, iv_a, iv_b, iv_c, dsts_a, dsts_b, dsts_c,
               sem_a, sem_b, sem_c, sem_a2, sem_b2, sem_c2,
               sem_a3, sem_b3, sem_c3, sem_ma, sem_mb, sem_mc):
    s = lax.axis_index("s")
    c = lax.axis_index("c")

    zero16 = jnp.zeros((16,), jnp.float32)

    @pl.loop(0, CHUNK)
    def _(r):
        for k in range(HALF // 16):
            rows_c[r, pl.ds(k * 16, 16)] = zero16

    row0 = s * ROW_STRIDE
    for j in range(ROWS_T // CHUNK):
        @pl.when(c == 0)
        def _():
            pltpu.sync_copy(init_lo_hbm.at[pl.ds(row0 + j * CHUNK, CHUNK)],
                            rows_a)

        @pl.when(c == 1)
        def _():
            pltpu.sync_copy(init_hi_hbm.at[pl.ds(row0 + j * CHUNK, CHUNK)],
                            rows_a)

        pltpu.sync_copy(rows_a, acc.at[pl.ds(row0 + j * CHUNK, CHUNK)])
    plsc.subcore_barrier()

    c0 = s * E_ITERS

    bufs_a = (m_a, rows_a, sem_a, iv_a, sem_a2, dsts_a, sem_a3, sem_ma)
    bufs_b = (m_b, rows_b, sem_b, iv_b, sem_b2, dsts_b, sem_b3, sem_mb)
    bufs_c = (m_c, rows_c, sem_c, iv_c, sem_c2, dsts_c, sem_c3, sem_mc)

    def start_gather(b):
        m_v, rows_v, sem, iv_v, sem2 = b[0], b[1], b[2], b[3], b[4]

        @pl.when(c == 0)
        def _():
            pltpu.async_copy(tbl_lo_hbm.at[m_v.at[0]], rows_v, sem)

        @pl.when(c == 1)
        def _():
            pltpu.async_copy(tbl_hi_hbm.at[m_v.at[0]], rows_v, sem)

        pltpu.async_copy(inv_hbm.at[m_v.at[1]], iv_v, sem2)

    def wait_gather(b):
        pltpu.make_async_copy(tbl_lo_hbm.at[b[0].at[0]], b[1], b[2]).wait()
        pltpu.make_async_copy(inv_hbm.at[b[0].at[1]], b[3], b[4]).wait()

    def copy_dsts(b):
        m_v, dsts_v = b[0], b[5]
        for k in range(CHUNK // 16):
            dsts_v[pl.ds(k * 16, 16)] = m_v[2, pl.ds(k * 16, 16)]

    def scale(b):
        rows_v, iv_v = b[1], b[3]

        @pl.loop(0, CHUNK // 16)
        def _(g):
            invs = iv_v[pl.ds(g * 16, 16)]
            for l in range(16):
                iv16 = invs[jnp.full((16,), l, jnp.int32)]
                j = g * 16 + l
                vals = [rows_v[j, pl.ds(k * 16, 16)]
                        for k in range(HALF // 16)]
                for k in range(HALF // 16):
                    rows_v[j, pl.ds(k * 16, 16)] = vals[k] * iv16

    def start_scatter(b):
        pltpu.async_copy(b[1], acc.at[b[5]], b[6], add=True)

    def wait_scatter(b):
        pltpu.make_async_copy(b[1], acc.at[b[5]], b[6]).wait()

    def start_meta(ci, b):
        pltpu.async_copy(meta_hbm.at[jnp.minimum(ci, TOT_CHUNKS - 1)],
                         b[0], b[7])

    def wait_meta(b):
        pltpu.make_async_copy(meta_hbm.at[0], b[0], b[7]).wait()

    def step(ci2, cur, nxt2):
        wait_scatter(nxt2)
        wait_meta(nxt2)
        start_gather(nxt2)
        wait_gather(cur)
        copy_dsts(cur)
        start_meta(ci2, cur)
        scale(cur)
        start_scatter(cur)

    pltpu.sync_copy(meta_hbm.at[c0], m_a)
    pltpu.sync_copy(meta_hbm.at[c0 + 1], m_b)
    pltpu.sync_copy(meta_hbm.at[c0], m_c)
    copy_dsts(bufs_c)
    start_gather(bufs_a)
    start_gather(bufs_b)
    start_scatter(bufs_c)
    start_meta(c0 + 2, bufs_c)

    @pl.loop(0, (E_ITERS - 1) // 3)
    def _(j):
        i = c0 + 3 * j
        step(i + 3, bufs_a, bufs_c)
        step(i + 4, bufs_b, bufs_a)
        step(i + 5, bufs_c, bufs_b)

    step(c0 + E_ITERS, bufs_a, bufs_c)

    wait_meta(bufs_a)
    wait_gather(bufs_b)
    wait_gather(bufs_c)
    wait_scatter(bufs_a)

    plsc.subcore_barrier()

    o0 = c * N_NODES + row0
    for j in range(ROWS_T // CHUNK):
        pltpu.sync_copy(acc.at[pl.ds(row0 + j * CHUNK, CHUNK)],
                        rows_a.at[pl.ds(0, CHUNK)])
        pltpu.sync_copy(rows_a.at[pl.ds(0, CHUNK)],
                        out_hbm.at[pl.ds(o0 + j * CHUNK, CHUNK)])


BLK = 1000
GRID = N_NODES // BLK


def _c1_body(p_ref, b0_ref, lw_ref, lb_ref,
             out0_ref, s_ref, m1_ref, m2_ref, m1_acc, m2_acc):
    i = pl.program_id(0)
    o = jnp.concatenate([p_ref[0], p_ref[1]], axis=-1) + b0_ref[...]
    out0_ref[...] = o
    logits = jnp.dot(o, lw_ref[...], preferred_element_type=jnp.float32) + lb_ref[...]
    m = jnp.max(logits, axis=-1, keepdims=True)
    e = jnp.exp(logits - m)
    sm = e / jnp.sum(e, axis=-1, keepdims=True)
    s_ref[...] = sm
    dn = (((0,), (0,)), ((), ()))
    pm1 = lax.dot_general(sm, o, dn, preferred_element_type=jnp.float32)
    pm2 = lax.dot_general(sm * sm, o * o, dn, preferred_element_type=jnp.float32)

    @pl.when(i == 0)
    def _():
        m1_acc[...] = jnp.zeros_like(m1_acc)
        m2_acc[...] = jnp.zeros_like(m2_acc)

    m1_acc[...] += pm1
    m2_acc[...] += pm2

    @pl.when(i == GRID - 1)
    def _():
        m1_ref[...] = m1_acc[...]
        m2_ref[...] = m2_acc[...]


_c1 = pl.pallas_call(
    _c1_body,
    grid=(GRID,),
    in_specs=[
        pl.BlockSpec((2, BLK, HALF), lambda i: (0, i, 0)),
        pl.BlockSpec((1, HID), lambda i: (0, 0)),
        pl.BlockSpec((HID, G_PAD), lambda i: (0, 0)),
        pl.BlockSpec((1, G_PAD), lambda i: (0, 0)),
    ],
    out_specs=[
        pl.BlockSpec((BLK, HID), lambda i: (i, 0)),
        pl.BlockSpec((BLK, G_PAD), lambda i: (i, 0)),
        pl.BlockSpec((G_PAD, HID), lambda i: (0, 0)),
        pl.BlockSpec((G_PAD, HID), lambda i: (0, 0)),
    ],
    out_shape=[
        jax.ShapeDtypeStruct((N_NODES, HID), jnp.float32),
        jax.ShapeDtypeStruct((N_NODES, G_PAD), jnp.float32),
        jax.ShapeDtypeStruct((G_PAD, HID), jnp.float32),
        jax.ShapeDtypeStruct((G_PAD, HID), jnp.float32),
    ],
    scratch_shapes=[
        pltpu.VMEM((G_PAD, HID), jnp.float32),
        pltpu.VMEM((G_PAD, HID), jnp.float32),
    ],
)


def _c2_body(o_ref, s_ref, m1_ref, m2_ref, g_ref, bb_ref, w1_ref, r1_ref, b1_ref,
             ylo_ref, yhi_ref, hrlo_ref, hrhi_ref):
    inv_n = 1.0 / N_NODES
    mu = m1_ref[...] * inv_n
    var = m2_ref[...] * inv_n - mu * mu
    a = g_ref[...] * lax.rsqrt(var + EPS)
    cst = jnp.sum(bb_ref[...] - mu * a, axis=0, keepdims=True)
    o = o_ref[...]
    sa = jnp.dot(s_ref[...], a, preferred_element_type=jnp.float32)
    h = jnp.maximum(o + LAMDA * (o * sa + cst), 0.0)
    hr = jnp.dot(h, r1_ref[...], preferred_element_type=jnp.float32) + b1_ref[...]
    hrlo_ref[...] = hr[:, :HALF]
    hrhi_ref[...] = hr[:, HALF:]
    for r in range(N_REL):
        yr = jnp.dot(h, w1_ref[r], preferred_element_type=jnp.float32)
        ylo_ref[r] = yr[:, :HALF]
        yhi_ref[r] = yr[:, HALF:]


_c2 = pl.pallas_call(
    _c2_body,
    grid=(GRID,),
    in_specs=[
        pl.BlockSpec((BLK, HID), lambda i: (i, 0)),
        pl.BlockSpec((BLK, G_PAD), lambda i: (i, 0)),
        pl.BlockSpec((G_PAD, HID), lambda i: (0, 0)),
        pl.BlockSpec((G_PAD, HID), lambda i: (0, 0)),
        pl.BlockSpec((G_PAD, HID), lambda i: (0, 0)),
        pl.BlockSpec((G_PAD, HID), lambda i: (0, 0)),
        pl.BlockSpec((N_REL, HID, HID), lambda i: (0, 0, 0)),
        pl.BlockSpec((HID, HID), lambda i: (0, 0)),
        pl.BlockSpec((1, HID), lambda i: (0, 0)),
    ],
    out_specs=[
        pl.BlockSpec((N_REL, BLK, HALF), lambda i: (0, i, 0)),
        pl.BlockSpec((N_REL, BLK, HALF), lambda i: (0, i, 0)),
        pl.BlockSpec((BLK, HALF), lambda i: (i, 0)),
        pl.BlockSpec((BLK, HALF), lambda i: (i, 0)),
    ],
    out_shape=[
        jax.ShapeDtypeStruct((N_REL, N_NODES, HALF), jnp.float32),
        jax.ShapeDtypeStruct((N_REL, N_NODES, HALF), jnp.float32),
        jax.ShapeDtypeStruct((N_NODES, HALF), jnp.float32),
        jax.ShapeDtypeStruct((N_NODES, HALF), jnp.float32),
    ],
)


def kernel(x, edge_index, edge_type, W0, root0, b0, lin_w, lin_b, bn_g, bn_b,
           W1, root1, b1):
    src = edge_index[0]
    dst = edge_index[1]
    et = edge_type
    gidx = et * N_NODES + src
    dseg = dst * N_REL + et

    npad = E_PAD_TOT - N_EDGES
    gidx_p = jnp.concatenate([gidx, jnp.zeros((npad,), jnp.int32)])
    dseg_p = jnp.concatenate([dseg, jnp.full((npad,), PAD_SEG, jnp.int32)])
    dst_p = jnp.concatenate([dst, jnp.zeros((npad,), jnp.int32)])
    meta = jnp.stack([gidx_p.reshape(TOT_CHUNKS, CHUNK),
                      dseg_p.reshape(TOT_CHUNKS, CHUNK),
                      dst_p.reshape(TOT_CHUNKS, CHUNK)], axis=1)

    inv = _count_inv(meta)

    w0f = W0.reshape(NR, HID)
    p0 = _edge_pass(w0f[:, :HALF], w0f[:, HALF:], meta, inv,
                    root0[:, :HALF], root0[:, HALF:])
    p0 = p0.reshape(NC, N_NODES, HALF)

    lw = jnp.pad(lin_w, ((0, 0), (0, G_PAD - N_GROUPS)))
    lb = jnp.concatenate(
        [lin_b, jnp.full((G_PAD - N_GROUPS,), -1e30, jnp.float32)]).reshape(1, G_PAD)
    out0, smat, m1, m2 = _c1(p0, b0.reshape(1, HID), lw, lb)

    gpad = jnp.pad(bn_g.reshape(N_GROUPS, HID), ((0, G_PAD - N_GROUPS), (0, 0)))
    bpad = jnp.pad(bn_b.reshape(N_GROUPS, HID), ((0, G_PAD - N_GROUPS), (0, 0)))
    ylo, yhi, hrlo, hrhi = _c2(out0, smat, m1, m2, gpad, bpad, W1, root1,
                               b1.reshape(1, HID))

    p1 = _edge_pass(ylo.reshape(NR, HALF), yhi.reshape(NR, HALF), meta, inv,
                    hrlo, hrhi)
    p1 = p1.reshape(NC, N_NODES, HALF)

    return jnp.concatenate([p1[0], p1[1]], axis=-1)


# speedup vs baseline: 1.0536x; 1.0536x over previous
import functools

import jax
import jax.numpy as jnp
from jax import lax
from jax.experimental import pallas as pl
from jax.experimental.pallas import tpu as pltpu
from jax.experimental.pallas import tpu_sc as plsc

N_NODES = 10000
N_EDGES = 320000
N_REL = 8
HID = 128
HALF = HID // 2
N_GROUPS = 3
G_PAD = 8
LAMDA = 0.01
EPS = 1e-5

NC = 2
NS = 16
NW = NC * NS

NR = N_NODES * N_REL
NR_PAD = 80384
SEG_W = NR_PAD // NW
SEG_T = NR_PAD // NS

CNT_E_TILE = N_EDGES // NS
CNT_CHUNK = 80
CNT_ITERS = CNT_E_TILE // CNT_CHUNK

CHUNK = 128
E_ITERS = 157
E_TILE_PAD = CHUNK * E_ITERS
E_PAD_TOT = NS * E_TILE_PAD
TOT_CHUNKS = E_PAD_TOT // CHUNK
PAD_SEG = NR_PAD - 1

ROW_STRIDE = 624
ROWS_T = 640

_sc_mesh = plsc.VectorSubcoreMesh(
    core_axis_name="c", subcore_axis_name="s", num_cores=NC, num_subcores=NS)
_sc_params = pltpu.CompilerParams(
    needs_layout_passes=False, use_tc_tiling_on_sc=False)


@functools.partial(
    pl.kernel,
    out_type=jax.ShapeDtypeStruct((NR_PAD,), jnp.float32),
    mesh=_sc_mesh,
    scratch_types=[
        pltpu.VMEM_SHARED((NR_PAD,), jnp.float32),
        pltpu.VMEM((3, CHUNK), jnp.int32),
        pltpu.VMEM((3, CHUNK), jnp.int32),
        pltpu.VMEM((3, CHUNK), jnp.int32),
        pltpu.VMEM((CHUNK,), jnp.float32),
        pltpu.VMEM((CHUNK,), jnp.float32),
        pltpu.VMEM((SEG_W,), jnp.float32),
        pltpu.SemaphoreType.DMA,
        pltpu.SemaphoreType.DMA,
        pltpu.SemaphoreType.DMA,
        pltpu.SemaphoreType.DMA,
        pltpu.SemaphoreType.DMA,
        pltpu.SemaphoreType.DMA,
    ],
    compiler_params=_sc_params,
)
def _count_inv(meta_hbm, inv_hbm, acc, m_a, m_b, m_c, ones_v, zeros_v, val_v,
               sem_la, sem_lb, sem_lc, sem_sa, sem_sb, sem_sc2):
    s = lax.axis_index("s")
    c = lax.axis_index("c")
    w = c * NS + s
    zero16 = jnp.zeros((16,), jnp.float32)
    one16 = jnp.ones((16,), jnp.float32)

    @pl.loop(0, CHUNK // 16)
    def _(i):
        ones_v[pl.ds(i * 16, 16)] = one16
        zeros_v[pl.ds(i * 16, 16)] = zero16

    @pl.loop(0, SEG_W // 16)
    def _(i):
        val_v[pl.ds(i * 16, 16)] = zero16

    pltpu.sync_copy(val_v, acc.at[pl.ds(s * SEG_T, SEG_W)])
    pltpu.sync_copy(val_v, acc.at[pl.ds(s * SEG_T + SEG_W, SEG_W)])
    plsc.subcore_barrier()

    c0 = s * E_ITERS

    def start_load(ci, m_v, sem_l):
        pltpu.async_copy(meta_hbm.at[jnp.minimum(ci, TOT_CHUNKS - 1)], m_v,
                         sem_l)

    def wait_load(ci, m_v, sem_l):
        pltpu.make_async_copy(meta_hbm.at[jnp.minimum(ci, TOT_CHUNKS - 1)],
                              m_v, sem_l).wait()

    def cstep(ci2, m_cur, sem_lc, sem_sc, m_nxt2, sem_ln, sem_sn):
        wait_load(0, m_cur, sem_lc)
        pltpu.async_copy(ones_v, acc.at[m_cur.at[1]], sem_sc, add=True)
        pltpu.make_async_copy(zeros_v, acc.at[m_nxt2.at[1]], sem_sn).wait()
        start_load(ci2, m_nxt2, sem_ln)

    pltpu.sync_copy(meta_hbm.at[c0], m_c)
    start_load(c0, m_a, sem_la)
    start_load(c0 + 1, m_b, sem_lb)
    pltpu.async_copy(zeros_v, acc.at[m_c.at[1]], sem_sc2, add=True)

    @pl.loop(0, (E_ITERS - 1) // 3)
    def _(j):
        i = c0 + 3 * j
        cstep(i + 2, m_a, sem_la, sem_sa, m_c, sem_lc, sem_sc2)
        cstep(i + 3, m_b, sem_lb, sem_sb, m_a, sem_la, sem_sa)
        cstep(i + 4, m_c, sem_lc, sem_sc2, m_b, sem_lb, sem_sb)

    cstep(c0 + E_ITERS, m_a, sem_la, sem_sa, m_c, sem_lc, sem_sc2)

    pltpu.make_async_copy(meta_hbm.at[0], m_b, sem_lb).wait()
    pltpu.make_async_copy(meta_hbm.at[0], m_c, sem_lc).wait()
    pltpu.make_async_copy(zeros_v, acc.at[m_a.at[1]], sem_sa).wait()

    plsc.subcore_barrier()
    pltpu.sync_copy(acc.at[pl.ds(w * SEG_W, SEG_W)], val_v)

    @pl.loop(0, SEG_W // 16)
    def _(i):
        v = val_v[pl.ds(i * 16, 16)]
        val_v[pl.ds(i * 16, 16)] = jnp.minimum(v, 1.0) / jnp.maximum(v, 1.0)

    @pl.when(w == NW - 1)
    def _():
        v = val_v[pl.ds(SEG_W - 16, 16)]
        val_v[pl.ds(SEG_W - 16, 16)] = jnp.where(
            lax.iota(jnp.int32, 16) < 15, v, 0.0)

    pltpu.sync_copy(val_v, inv_hbm.at[pl.ds(w * SEG_W, SEG_W)])


@functools.partial(
    pl.kernel,
    out_type=jax.ShapeDtypeStruct((NC * N_NODES, HALF), jnp.float32),
    mesh=_sc_mesh,
    scratch_types=[
        pltpu.VMEM_SHARED((N_NODES, HALF), jnp.float32),
        pltpu.VMEM((CHUNK, HALF), jnp.float32),
        pltpu.VMEM((CHUNK, HALF), jnp.float32),
        pltpu.VMEM((CHUNK, HALF), jnp.float32),
        pltpu.VMEM((3, CHUNK), jnp.int32),
        pltpu.VMEM((3, CHUNK), jnp.int32),
        pltpu.VMEM((3, CHUNK), jnp.int32),
        pltpu.VMEM((CHUNK,), jnp.float32),
        pltpu.VMEM((CHUNK,), jnp.float32),
        pltpu.VMEM((CHUNK,), jnp.float32),
        pltpu.VMEM((CHUNK,), jnp.int32),
        pltpu.VMEM((CHUNK,), jnp.int32),
        pltpu.VMEM((CHUNK,), jnp.int32),
        pltpu.SemaphoreType.DMA,
        pltpu.SemaphoreType.DMA,
        pltpu.SemaphoreType.DMA,
        pltpu.SemaphoreType.DMA,
        pltpu.SemaphoreType.DMA,
        pltpu.SemaphoreType.DMA,
        pltpu.SemaphoreType.DMA,
        pltpu.SemaphoreType.DMA,
        pltpu.SemaphoreType.DMA,
        pltpu.SemaphoreType.DMA,
        pltpu.SemaphoreType.DMA,
        pltpu.SemaphoreType.DMA,
    ],
    compiler_params=_sc_params,
)
def _edge_pass(tbl_lo_hbm, tbl_hi_hbm, meta_hbm, inv_hbm,
               out_hbm, acc, rows_a, rows_b, rows_c, m_a, m_b,
               m_c, iv_a, iv_b, iv_c, dsts_a, dsts_b, dsts_c,
               sem_a, sem_b, sem_c, sem_a2, sem_b2, sem_c2,
               sem_a3, sem_b3, sem_c3, sem_ma, sem_mb, sem_mc):
    s = lax.axis_index("s")
    c = lax.axis_index("c")

    zero16 = jnp.zeros((16,), jnp.float32)

    @pl.loop(0, CHUNK)
    def _(r):
        for k in range(HALF // 16):
            rows_a[r, pl.ds(k * 16, 16)] = zero16
            rows_c[r, pl.ds(k * 16, 16)] = zero16

    row0 = s * ROW_STRIDE
    for j in range(ROWS_T // CHUNK):
        pltpu.sync_copy(rows_a.at[pl.ds(0, CHUNK)],
                        acc.at[pl.ds(row0 + j * CHUNK, CHUNK)])
    plsc.subcore_barrier()

    c0 = s * E_ITERS

    bufs_a = (m_a, rows_a, sem_a, iv_a, sem_a2, dsts_a, sem_a3, sem_ma)
    bufs_b = (m_b, rows_b, sem_b, iv_b, sem_b2, dsts_b, sem_b3, sem_mb)
    bufs_c = (m_c, rows_c, sem_c, iv_c, sem_c2, dsts_c, sem_c3, sem_mc)

    def start_gather(b):
        m_v, rows_v, sem, iv_v, sem2 = b[0], b[1], b[2], b[3], b[4]

        @pl.when(c == 0)
        def _():
            pltpu.async_copy(tbl_lo_hbm.at[m_v.at[0]], rows_v, sem)

        @pl.when(c == 1)
        def _():
            pltpu.async_copy(tbl_hi_hbm.at[m_v.at[0]], rows_v, sem)

        pltpu.async_copy(inv_hbm.at[m_v.at[1]], iv_v, sem2)

    def wait_gather(b):
        pltpu.make_async_copy(tbl_lo_hbm.at[b[0].at[0]], b[1], b[2]).wait()
        pltpu.make_async_copy(inv_hbm.at[b[0].at[1]], b[3], b[4]).wait()

    def copy_dsts(b):
        m_v, dsts_v = b[0], b[5]
        for k in range(CHUNK // 16):
            dsts_v[pl.ds(k * 16, 16)] = m_v[2, pl.ds(k * 16, 16)]

    def scale(b):
        rows_v, iv_v = b[1], b[3]

        @pl.loop(0, CHUNK // 16)
        def _(g):
            invs = iv_v[pl.ds(g * 16, 16)]
            for l in range(16):
                iv16 = invs[jnp.full((16,), l, jnp.int32)]
                j = g * 16 + l
                vals = [rows_v[j, pl.ds(k * 16, 16)]
                        for k in range(HALF // 16)]
                for k in range(HALF // 16):
                    rows_v[j, pl.ds(k * 16, 16)] = vals[k] * iv16

    def start_scatter(b):
        pltpu.async_copy(b[1], acc.at[b[5]], b[6], add=True)

    def wait_scatter(b):
        pltpu.make_async_copy(b[1], acc.at[b[5]], b[6]).wait()

    def start_meta(ci, b):
        pltpu.async_copy(meta_hbm.at[jnp.minimum(ci, TOT_CHUNKS - 1)],
                         b[0], b[7])

    def wait_meta(b):
        pltpu.make_async_copy(meta_hbm.at[0], b[0], b[7]).wait()

    def step(ci2, cur, nxt2):
        wait_scatter(nxt2)
        wait_meta(nxt2)
        start_gather(nxt2)
        wait_gather(cur)
        copy_dsts(cur)
        start_meta(ci2, cur)
        scale(cur)
        start_scatter(cur)

    pltpu.sync_copy(meta_hbm.at[c0], m_a)
    pltpu.sync_copy(meta_hbm.at[c0 + 1], m_b)
    pltpu.sync_copy(meta_hbm.at[c0], m_c)
    copy_dsts(bufs_c)
    start_gather(bufs_a)
    start_gather(bufs_b)
    start_scatter(bufs_c)
    start_meta(c0 + 2, bufs_c)

    @pl.loop(0, (E_ITERS - 1) // 3)
    def _(j):
        i = c0 + 3 * j
        step(i + 3, bufs_a, bufs_c)
        step(i + 4, bufs_b, bufs_a)
        step(i + 5, bufs_c, bufs_b)

    step(c0 + E_ITERS, bufs_a, bufs_c)

    wait_meta(bufs_a)
    wait_gather(bufs_b)
    wait_gather(bufs_c)
    wait_scatter(bufs_a)

    plsc.subcore_barrier()

    o0 = c * N_NODES + row0
    for j in range(ROWS_T // CHUNK):
        pltpu.sync_copy(acc.at[pl.ds(row0 + j * CHUNK, CHUNK)],
                        rows_a.at[pl.ds(0, CHUNK)])
        pltpu.sync_copy(rows_a.at[pl.ds(0, CHUNK)],
                        out_hbm.at[pl.ds(o0 + j * CHUNK, CHUNK)])


BLK = 1000
GRID = N_NODES // BLK


def _c1_body(p_ref, r0_ref, b0_ref, lw_ref, lb_ref,
             out0_ref, s_ref, m1_ref, m2_ref, m1_acc, m2_acc):
    i = pl.program_id(0)
    o = (jnp.concatenate([p_ref[0], p_ref[1]], axis=-1) + r0_ref[...]
         + b0_ref[...])
    out0_ref[...] = o
    logits = jnp.dot(o, lw_ref[...], preferred_element_type=jnp.float32) + lb_ref[...]
    m = jnp.max(logits, axis=-1, keepdims=True)
    e = jnp.exp(logits - m)
    sm = e / jnp.sum(e, axis=-1, keepdims=True)
    s_ref[...] = sm
    dn = (((0,), (0,)), ((), ()))
    pm1 = lax.dot_general(sm, o, dn, preferred_element_type=jnp.float32)
    pm2 = lax.dot_general(sm * sm, o * o, dn, preferred_element_type=jnp.float32)

    @pl.when(i == 0)
    def _():
        m1_acc[...] = jnp.zeros_like(m1_acc)
        m2_acc[...] = jnp.zeros_like(m2_acc)

    m1_acc[...] += pm1
    m2_acc[...] += pm2

    @pl.when(i == GRID - 1)
    def _():
        m1_ref[...] = m1_acc[...]
        m2_ref[...] = m2_acc[...]


_c1 = pl.pallas_call(
    _c1_body,
    grid=(GRID,),
    in_specs=[
        pl.BlockSpec((2, BLK, HALF), lambda i: (0, i, 0)),
        pl.BlockSpec((BLK, HID), lambda i: (i, 0)),
        pl.BlockSpec((1, HID), lambda i: (0, 0)),
        pl.BlockSpec((HID, G_PAD), lambda i: (0, 0)),
        pl.BlockSpec((1, G_PAD), lambda i: (0, 0)),
    ],
    out_specs=[
        pl.BlockSpec((BLK, HID), lambda i: (i, 0)),
        pl.BlockSpec((BLK, G_PAD), lambda i: (i, 0)),
        pl.BlockSpec((G_PAD, HID), lambda i: (0, 0)),
        pl.BlockSpec((G_PAD, HID), lambda i: (0, 0)),
    ],
    out_shape=[
        jax.ShapeDtypeStruct((N_NODES, HID), jnp.float32),
        jax.ShapeDtypeStruct((N_NODES, G_PAD), jnp.float32),
        jax.ShapeDtypeStruct((G_PAD, HID), jnp.float32),
        jax.ShapeDtypeStruct((G_PAD, HID), jnp.float32),
    ],
    scratch_shapes=[
        pltpu.VMEM((G_PAD, HID), jnp.float32),
        pltpu.VMEM((G_PAD, HID), jnp.float32),
    ],
)


def _c2_body(o_ref, s_ref, m1_ref, m2_ref, g_ref, bb_ref, w1_ref, r1_ref, b1_ref,
             ylo_ref, yhi_ref, hr_ref):
    inv_n = 1.0 / N_NODES
    mu = m1_ref[...] * inv_n
    var = m2_ref[...] * inv_n - mu * mu
    a = g_ref[...] * lax.rsqrt(var + EPS)
    cst = jnp.sum(bb_ref[...] - mu * a, axis=0, keepdims=True)
    o = o_ref[...]
    sa = jnp.dot(s_ref[...], a, preferred_element_type=jnp.float32)
    h = jnp.maximum(o + LAMDA * (o * sa + cst), 0.0)
    hr_ref[...] = jnp.dot(h, r1_ref[...], preferred_element_type=jnp.float32) + b1_ref[...]
    for r in range(N_REL):
        yr = jnp.dot(h, w1_ref[r], preferred_element_type=jnp.float32)
        ylo_ref[r] = yr[:, :HALF]
        yhi_ref[r] = yr[:, HALF:]


_c2 = pl.pallas_call(
    _c2_body,
    grid=(GRID,),
    in_specs=[
        pl.BlockSpec((BLK, HID), lambda i: (i, 0)),
        pl.BlockSpec((BLK, G_PAD), lambda i: (i, 0)),
        pl.BlockSpec((G_PAD, HID), lambda i: (0, 0)),
        pl.BlockSpec((G_PAD, HID), lambda i: (0, 0)),
        pl.BlockSpec((G_PAD, HID), lambda i: (0, 0)),
        pl.BlockSpec((G_PAD, HID), lambda i: (0, 0)),
        pl.BlockSpec((N_REL, HID, HID), lambda i: (0, 0, 0)),
        pl.BlockSpec((HID, HID), lambda i: (0, 0)),
        pl.BlockSpec((1, HID), lambda i: (0, 0)),
    ],
    out_specs=[
        pl.BlockSpec((N_REL, BLK, HALF), lambda i: (0, i, 0)),
        pl.BlockSpec((N_REL, BLK, HALF), lambda i: (0, i, 0)),
        pl.BlockSpec((BLK, HID), lambda i: (i, 0)),
    ],
    out_shape=[
        jax.ShapeDtypeStruct((N_REL, N_NODES, HALF), jnp.float32),
        jax.ShapeDtypeStruct((N_REL, N_NODES, HALF), jnp.float32),
        jax.ShapeDtypeStruct((N_NODES, HID), jnp.float32),
    ],
)


def _fin_body(p_ref, hr_ref, out_ref):
    out_ref[...] = jnp.concatenate([p_ref[0], p_ref[1]], axis=-1) + hr_ref[...]


_fin = pl.pallas_call(
    _fin_body,
    grid=(GRID,),
    in_specs=[
        pl.BlockSpec((2, BLK, HALF), lambda i: (0, i, 0)),
        pl.BlockSpec((BLK, HID), lambda i: (i, 0)),
    ],
    out_specs=pl.BlockSpec((BLK, HID), lambda i: (i, 0)),
    out_shape=jax.ShapeDtypeStruct((N_NODES, HID), jnp.float32),
)


def kernel(x, edge_index, edge_type, W0, root0, b0, lin_w, lin_b, bn_g, bn_b,
           W1, root1, b1):
    src = edge_index[0]
    dst = edge_index[1]
    et = edge_type
    gidx = et * N_NODES + src
    dseg = dst * N_REL + et

    npad = E_PAD_TOT - N_EDGES
    gidx_p = jnp.concatenate([gidx, jnp.zeros((npad,), jnp.int32)])
    dseg_p = jnp.concatenate([dseg, jnp.full((npad,), PAD_SEG, jnp.int32)])
    dst_p = jnp.concatenate([dst, jnp.zeros((npad,), jnp.int32)])
    meta = jnp.stack([gidx_p.reshape(TOT_CHUNKS, CHUNK),
                      dseg_p.reshape(TOT_CHUNKS, CHUNK),
                      dst_p.reshape(TOT_CHUNKS, CHUNK)], axis=1)

    inv = _count_inv(meta)

    w0f = W0.reshape(NR, HID)
    p0 = _edge_pass(w0f[:, :HALF], w0f[:, HALF:], meta, inv)
    p0 = p0.reshape(NC, N_NODES, HALF)

    lw = jnp.pad(lin_w, ((0, 0), (0, G_PAD - N_GROUPS)))
    lb = jnp.concatenate(
        [lin_b, jnp.full((G_PAD - N_GROUPS,), -1e30, jnp.float32)]).reshape(1, G_PAD)
    out0, smat, m1, m2 = _c1(p0, root0, b0.reshape(1, HID), lw, lb)

    gpad = jnp.pad(bn_g.reshape(N_GROUPS, HID), ((0, G_PAD - N_GROUPS), (0, 0)))
    bpad = jnp.pad(bn_b.reshape(N_GROUPS, HID), ((0, G_PAD - N_GROUPS), (0, 0)))
    ylo, yhi, hroot = _c2(out0, smat, m1, m2, gpad, bpad, W1, root1,
                          b1.reshape(1, HID))

    p1 = _edge_pass(ylo.reshape(NR, HALF), yhi.reshape(NR, HALF), meta, inv)
    p1 = p1.reshape(NC, N_NODES, HALF)

    return _fin(p1, hroot)


# speedup vs baseline: 1.0539x; 1.0003x over previous
import functools

import jax
import jax.numpy as jnp
from jax import lax
from jax.experimental import pallas as pl
from jax.experimental.pallas import tpu as pltpu
from jax.experimental.pallas import tpu_sc as plsc

N_NODES = 10000
N_EDGES = 320000
N_REL = 8
HID = 128
HALF = HID // 2
N_GROUPS = 3
G_PAD = 8
LAMDA = 0.01
EPS = 1e-5

NC = 2
NS = 16
NW = NC * NS

NR = N_NODES * N_REL
NR_PAD = 80384
SEG_W = NR_PAD // NW
SEG_T = NR_PAD // NS

CHUNK = 128
E_ITERS = 157
E_TILE_PAD = CHUNK * E_ITERS
E_PAD_TOT = NS * E_TILE_PAD
TOT_CHUNKS = E_PAD_TOT // CHUNK
PAD_SEG = NR_PAD - 1

ROW_STRIDE = 624
ROWS_T = 640

_sc_mesh = plsc.VectorSubcoreMesh(
    core_axis_name="c", subcore_axis_name="s", num_cores=NC, num_subcores=NS)
_sc_params = pltpu.CompilerParams(
    needs_layout_passes=False, use_tc_tiling_on_sc=False)


@functools.partial(
    pl.kernel,
    out_type=jax.ShapeDtypeStruct((NR_PAD,), jnp.float32),
    mesh=_sc_mesh,
    scratch_types=[
        pltpu.VMEM_SHARED((NR_PAD,), jnp.float32),
        pltpu.VMEM((3, CHUNK), jnp.int32),
        pltpu.VMEM((3, CHUNK), jnp.int32),
        pltpu.VMEM((3, CHUNK), jnp.int32),
        pltpu.VMEM((CHUNK,), jnp.float32),
        pltpu.VMEM((CHUNK,), jnp.float32),
        pltpu.VMEM((SEG_W,), jnp.float32),
        pltpu.SemaphoreType.DMA,
        pltpu.SemaphoreType.DMA,
        pltpu.SemaphoreType.DMA,
        pltpu.SemaphoreType.DMA,
        pltpu.SemaphoreType.DMA,
        pltpu.SemaphoreType.DMA,
    ],
    compiler_params=_sc_params,
)
def _count_inv(meta_hbm, inv_hbm, acc, m_a, m_b, m_c, ones_v, zeros_v, val_v,
               sem_la, sem_lb, sem_lc, sem_sa, sem_sb, sem_sc2):
    s = lax.axis_index("s")
    c = lax.axis_index("c")
    w = c * NS + s
    zero16 = jnp.zeros((16,), jnp.float32)
    one16 = jnp.ones((16,), jnp.float32)

    @pl.loop(0, CHUNK // 16)
    def _(i):
        ones_v[pl.ds(i * 16, 16)] = one16
        zeros_v[pl.ds(i * 16, 16)] = zero16

    @pl.loop(0, SEG_W // 16)
    def _(i):
        val_v[pl.ds(i * 16, 16)] = zero16

    pltpu.sync_copy(val_v, acc.at[pl.ds(s * SEG_T, SEG_W)])
    pltpu.sync_copy(val_v, acc.at[pl.ds(s * SEG_T + SEG_W, SEG_W)])
    plsc.subcore_barrier()

    c0 = s * E_ITERS

    def start_load(ci, m_v, sem_l):
        pltpu.async_copy(meta_hbm.at[jnp.minimum(ci, TOT_CHUNKS - 1)], m_v,
                         sem_l)

    def wait_load(ci, m_v, sem_l):
        pltpu.make_async_copy(meta_hbm.at[jnp.minimum(ci, TOT_CHUNKS - 1)],
                              m_v, sem_l).wait()

    def cstep(ci2, m_cur, sem_lc, sem_sc, m_nxt2, sem_ln, sem_sn):
        wait_load(0, m_cur, sem_lc)
        pltpu.async_copy(ones_v, acc.at[m_cur.at[1]], sem_sc, add=True)
        pltpu.make_async_copy(zeros_v, acc.at[m_nxt2.at[1]], sem_sn).wait()
        start_load(ci2, m_nxt2, sem_ln)

    pltpu.sync_copy(meta_hbm.at[c0], m_c)
    start_load(c0, m_a, sem_la)
    start_load(c0 + 1, m_b, sem_lb)
    pltpu.async_copy(zeros_v, acc.at[m_c.at[1]], sem_sc2, add=True)

    @pl.loop(0, (E_ITERS - 1) // 3)
    def _(j):
        i = c0 + 3 * j
        cstep(i + 2, m_a, sem_la, sem_sa, m_c, sem_lc, sem_sc2)
        cstep(i + 3, m_b, sem_lb, sem_sb, m_a, sem_la, sem_sa)
        cstep(i + 4, m_c, sem_lc, sem_sc2, m_b, sem_lb, sem_sb)

    cstep(c0 + E_ITERS, m_a, sem_la, sem_sa, m_c, sem_lc, sem_sc2)

    pltpu.make_async_copy(meta_hbm.at[0], m_b, sem_lb).wait()
    pltpu.make_async_copy(meta_hbm.at[0], m_c, sem_lc).wait()
    pltpu.make_async_copy(zeros_v, acc.at[m_a.at[1]], sem_sa).wait()

    plsc.subcore_barrier()
    pltpu.sync_copy(acc.at[pl.ds(w * SEG_W, SEG_W)], val_v)

    @pl.loop(0, SEG_W // 16)
    def _(i):
        v = val_v[pl.ds(i * 16, 16)]
        val_v[pl.ds(i * 16, 16)] = jnp.minimum(v, 1.0) / jnp.maximum(v, 1.0)

    @pl.when(w == NW - 1)
    def _():
        v = val_v[pl.ds(SEG_W - 16, 16)]
        val_v[pl.ds(SEG_W - 16, 16)] = jnp.where(
            lax.iota(jnp.int32, 16) < 15, v, 0.0)

    pltpu.sync_copy(val_v, inv_hbm.at[pl.ds(w * SEG_W, SEG_W)])


@functools.partial(
    pl.kernel,
    out_type=jax.ShapeDtypeStruct((NC * N_NODES, HALF), jnp.float32),
    mesh=_sc_mesh,
    scratch_types=[
        pltpu.VMEM_SHARED((N_NODES, HALF), jnp.float32),
        pltpu.VMEM((CHUNK, HALF), jnp.float32),
        pltpu.VMEM((CHUNK, HALF), jnp.float32),
        pltpu.VMEM((CHUNK, HALF), jnp.float32),
        pltpu.VMEM((3, CHUNK), jnp.int32),
        pltpu.VMEM((3, CHUNK), jnp.int32),
        pltpu.VMEM((3, CHUNK), jnp.int32),
        pltpu.VMEM((CHUNK,), jnp.float32),
        pltpu.VMEM((CHUNK,), jnp.float32),
        pltpu.VMEM((CHUNK,), jnp.float32),
        pltpu.VMEM((CHUNK,), jnp.int32),
        pltpu.VMEM((CHUNK,), jnp.int32),
        pltpu.VMEM((CHUNK,), jnp.int32),
        pltpu.SemaphoreType.DMA,
        pltpu.SemaphoreType.DMA,
        pltpu.SemaphoreType.DMA,
        pltpu.SemaphoreType.DMA,
        pltpu.SemaphoreType.DMA,
        pltpu.SemaphoreType.DMA,
        pltpu.SemaphoreType.DMA,
        pltpu.SemaphoreType.DMA,
        pltpu.SemaphoreType.DMA,
        pltpu.SemaphoreType.DMA,
        pltpu.SemaphoreType.DMA,
        pltpu.SemaphoreType.DMA,
    ],
    compiler_params=_sc_params,
)
def _edge_pass(tbl_lo_hbm, tbl_hi_hbm, meta_hbm, inv_hbm,
               out_hbm, acc, rows_a, rows_b, rows_c, m_a, m_b,
               m_c, iv_a, iv_b, iv_c, dsts_a, dsts_b, dsts_c,
               sem_a, sem_b, sem_c, sem_a2, sem_b2, sem_c2,
               sem_a3, sem_b3, sem_c3, sem_ma, sem_mb, sem_mc):
    s = lax.axis_index("s")
    c = lax.axis_index("c")

    zero16 = jnp.zeros((16,), jnp.float32)

    @pl.loop(0, CHUNK)
    def _(r):
        for k in range(HALF // 16):
            rows_a[r, pl.ds(k * 16, 16)] = zero16
            rows_c[r, pl.ds(k * 16, 16)] = zero16

    row0 = s * ROW_STRIDE
    for j in range(ROWS_T // CHUNK):
        pltpu.sync_copy(rows_a.at[pl.ds(0, CHUNK)],
                        acc.at[pl.ds(row0 + j * CHUNK, CHUNK)])
    plsc.subcore_barrier()

    c0 = s * E_ITERS

    bufs_a = (m_a, rows_a, sem_a, iv_a, sem_a2, dsts_a, sem_a3, sem_ma)
    bufs_b = (m_b, rows_b, sem_b, iv_b, sem_b2, dsts_b, sem_b3, sem_mb)
    bufs_c = (m_c, rows_c, sem_c, iv_c, sem_c2, dsts_c, sem_c3, sem_mc)

    def start_gather(b):
        m_v, rows_v, sem, iv_v, sem2 = b[0], b[1], b[2], b[3], b[4]

        @pl.when(c == 0)
        def _():
            pltpu.async_copy(tbl_lo_hbm.at[m_v.at[0]], rows_v, sem)

        @pl.when(c == 1)
        def _():
            pltpu.async_copy(tbl_hi_hbm.at[m_v.at[0]], rows_v, sem)

        pltpu.async_copy(inv_hbm.at[m_v.at[1]], iv_v, sem2)

    def wait_gather(b):
        pltpu.make_async_copy(tbl_lo_hbm.at[b[0].at[0]], b[1], b[2]).wait()
        pltpu.make_async_copy(inv_hbm.at[b[0].at[1]], b[3], b[4]).wait()

    def copy_dsts(b):
        m_v, dsts_v = b[0], b[5]
        for k in range(CHUNK // 16):
            dsts_v[pl.ds(k * 16, 16)] = m_v[2, pl.ds(k * 16, 16)]

    def scale(b):
        rows_v, iv_v = b[1], b[3]

        @pl.loop(0, CHUNK // 16)
        def _(g):
            invs = iv_v[pl.ds(g * 16, 16)]
            for l in range(16):
                iv16 = invs[jnp.full((16,), l, jnp.int32)]
                j = g * 16 + l
                vals = [rows_v[j, pl.ds(k * 16, 16)]
                        for k in range(HALF // 16)]
                for k in range(HALF // 16):
                    rows_v[j, pl.ds(k * 16, 16)] = vals[k] * iv16

    def start_scatter(b):
        pltpu.async_copy(b[1], acc.at[b[5]], b[6], add=True)

    def wait_scatter(b):
        pltpu.make_async_copy(b[1], acc.at[b[5]], b[6]).wait()

    def start_meta(ci, b):
        pltpu.async_copy(meta_hbm.at[jnp.minimum(ci, TOT_CHUNKS - 1)],
                         b[0], b[7])

    def wait_meta(b):
        pltpu.make_async_copy(meta_hbm.at[0], b[0], b[7]).wait()

    def step(ci2, cur, nxt2):
        wait_scatter(nxt2)
        wait_meta(nxt2)
        start_gather(nxt2)
        wait_gather(cur)
        copy_dsts(cur)
        start_meta(ci2, cur)
        scale(cur)
        start_scatter(cur)

    pltpu.sync_copy(meta_hbm.at[c0], m_a)
    pltpu.sync_copy(meta_hbm.at[c0 + 1], m_b)
    pltpu.sync_copy(meta_hbm.at[c0], m_c)
    copy_dsts(bufs_c)
    start_gather(bufs_a)
    start_gather(bufs_b)
    start_scatter(bufs_c)
    start_meta(c0 + 2, bufs_c)

    @pl.loop(0, (E_ITERS - 1) // 3)
    def _(j):
        i = c0 + 3 * j
        step(i + 3, bufs_a, bufs_c)
        step(i + 4, bufs_b, bufs_a)
        step(i + 5, bufs_c, bufs_b)

    step(c0 + E_ITERS, bufs_a, bufs_c)

    wait_meta(bufs_a)
    wait_gather(bufs_b)
    wait_gather(bufs_c)
    wait_scatter(bufs_a)

    plsc.subcore_barrier()

    o0 = c * N_NODES + row0
    for j in range(ROWS_T // CHUNK):
        pltpu.sync_copy(acc.at[pl.ds(row0 + j * CHUNK, CHUNK)],
                        rows_a.at[pl.ds(0, CHUNK)])
        pltpu.sync_copy(rows_a.at[pl.ds(0, CHUNK)],
                        out_hbm.at[pl.ds(o0 + j * CHUNK, CHUNK)])


BLK = 1000
GRID = N_NODES // BLK


def _c1_body(p_ref, r0_ref, b0_ref, lw_ref, lb_ref,
             out0_ref, s_ref, m1_ref, m2_ref, m1_acc, m2_acc):
    i = pl.program_id(0)
    o = (jnp.concatenate([p_ref[0], p_ref[1]], axis=-1) + r0_ref[...]
         + b0_ref[...])
    out0_ref[...] = o
    logits = jnp.dot(o, lw_ref[...], preferred_element_type=jnp.float32) + lb_ref[...]
    m = jnp.max(logits, axis=-1, keepdims=True)
    e = jnp.exp(logits - m)
    sm = e / jnp.sum(e, axis=-1, keepdims=True)
    s_ref[...] = sm
    dn = (((0,), (0,)), ((), ()))
    pm1 = lax.dot_general(sm, o, dn, preferred_element_type=jnp.float32)
    pm2 = lax.dot_general(sm * sm, o * o, dn, preferred_element_type=jnp.float32)

    @pl.when(i == 0)
    def _():
        m1_acc[...] = jnp.zeros_like(m1_acc)
        m2_acc[...] = jnp.zeros_like(m2_acc)

    m1_acc[...] += pm1
    m2_acc[...] += pm2

    @pl.when(i == GRID - 1)
    def _():
        m1_ref[...] = m1_acc[...]
        m2_ref[...] = m2_acc[...]


_c1 = pl.pallas_call(
    _c1_body,
    grid=(GRID,),
    in_specs=[
        pl.BlockSpec((2, BLK, HALF), lambda i: (0, i, 0)),
        pl.BlockSpec((BLK, HID), lambda i: (i, 0)),
        pl.BlockSpec((1, HID), lambda i: (0, 0)),
        pl.BlockSpec((HID, G_PAD), lambda i: (0, 0)),
        pl.BlockSpec((1, G_PAD), lambda i: (0, 0)),
    ],
    out_specs=[
        pl.BlockSpec((BLK, HID), lambda i: (i, 0)),
        pl.BlockSpec((BLK, G_PAD), lambda i: (i, 0)),
        pl.BlockSpec((G_PAD, HID), lambda i: (0, 0)),
        pl.BlockSpec((G_PAD, HID), lambda i: (0, 0)),
    ],
    out_shape=[
        jax.ShapeDtypeStruct((N_NODES, HID), jnp.float32),
        jax.ShapeDtypeStruct((N_NODES, G_PAD), jnp.float32),
        jax.ShapeDtypeStruct((G_PAD, HID), jnp.float32),
        jax.ShapeDtypeStruct((G_PAD, HID), jnp.float32),
    ],
    scratch_shapes=[
        pltpu.VMEM((G_PAD, HID), jnp.float32),
        pltpu.VMEM((G_PAD, HID), jnp.float32),
    ],
)


def _c2_body(o_ref, s_ref, m1_ref, m2_ref, g_ref, bb_ref, w1_ref, r1_ref, b1_ref,
             ylo_ref, yhi_ref, hr_ref):
    inv_n = 1.0 / N_NODES
    mu = m1_ref[...] * inv_n
    var = m2_ref[...] * inv_n - mu * mu
    a = g_ref[...] * lax.rsqrt(var + EPS)
    cst = jnp.sum(bb_ref[...] - mu * a, axis=0, keepdims=True)
    o = o_ref[...]
    sa = jnp.dot(s_ref[...], a, preferred_element_type=jnp.float32)
    h = jnp.maximum(o + LAMDA * (o * sa + cst), 0.0)
    hr_ref[...] = jnp.dot(h, r1_ref[...], preferred_element_type=jnp.float32) + b1_ref[...]
    for r in range(N_REL):
        yr = jnp.dot(h, w1_ref[r], preferred_element_type=jnp.float32)
        ylo_ref[r] = yr[:, :HALF]
        yhi_ref[r] = yr[:, HALF:]


_c2 = pl.pallas_call(
    _c2_body,
    grid=(GRID,),
    in_specs=[
        pl.BlockSpec((BLK, HID), lambda i: (i, 0)),
        pl.BlockSpec((BLK, G_PAD), lambda i: (i, 0)),
        pl.BlockSpec((G_PAD, HID), lambda i: (0, 0)),
        pl.BlockSpec((G_PAD, HID), lambda i: (0, 0)),
        pl.BlockSpec((G_PAD, HID), lambda i: (0, 0)),
        pl.BlockSpec((G_PAD, HID), lambda i: (0, 0)),
        pl.BlockSpec((N_REL, HID, HID), lambda i: (0, 0, 0)),
        pl.BlockSpec((HID, HID), lambda i: (0, 0)),
        pl.BlockSpec((1, HID), lambda i: (0, 0)),
    ],
    out_specs=[
        pl.BlockSpec((N_REL, BLK, HALF), lambda i: (0, i, 0)),
        pl.BlockSpec((N_REL, BLK, HALF), lambda i: (0, i, 0)),
        pl.BlockSpec((BLK, HID), lambda i: (i, 0)),
    ],
    out_shape=[
        jax.ShapeDtypeStruct((N_REL, N_NODES, HALF), jnp.float32),
        jax.ShapeDtypeStruct((N_REL, N_NODES, HALF), jnp.float32),
        jax.ShapeDtypeStruct((N_NODES, HID), jnp.float32),
    ],
)


def _fin_body(p_ref, hr_ref, out_ref):
    out_ref[...] = jnp.concatenate([p_ref[0], p_ref[1]], axis=-1) + hr_ref[...]


_fin = pl.pallas_call(
    _fin_body,
    grid=(GRID,),
    in_specs=[
        pl.BlockSpec((2, BLK, HALF), lambda i: (0, i, 0)),
        pl.BlockSpec((BLK, HID), lambda i: (i, 0)),
    ],
    out_specs=pl.BlockSpec((BLK, HID), lambda i: (i, 0)),
    out_shape=jax.ShapeDtypeStruct((N_NODES, HID), jnp.float32),
)


def kernel(x, edge_index, edge_type, W0, root0, b0, lin_w, lin_b, bn_g, bn_b,
           W1, root1, b1):
    src = edge_index[0]
    dst = edge_index[1]
    et = edge_type
    gidx = et * N_NODES + src
    dseg = dst * N_REL + et

    npad = E_PAD_TOT - N_EDGES
    gidx_p = jnp.concatenate([gidx, jnp.zeros((npad,), jnp.int32)])
    dseg_p = jnp.concatenate([dseg, jnp.full((npad,), PAD_SEG, jnp.int32)])
    dst_p = jnp.concatenate([dst, jnp.zeros((npad,), jnp.int32)])
    meta = jnp.stack([gidx_p.reshape(TOT_CHUNKS, CHUNK),
                      dseg_p.reshape(TOT_CHUNKS, CHUNK),
                      dst_p.reshape(TOT_CHUNKS, CHUNK)], axis=1)

    inv = _count_inv(meta)

    w0f = W0.reshape(NR, HID)
    p0 = _edge_pass(w0f[:, :HALF], w0f[:, HALF:], meta, inv)
    p0 = p0.reshape(NC, N_NODES, HALF)

    lw = jnp.pad(lin_w, ((0, 0), (0, G_PAD - N_GROUPS)))
    lb = jnp.concatenate(
        [lin_b, jnp.full((G_PAD - N_GROUPS,), -1e30, jnp.float32)]).reshape(1, G_PAD)
    out0, smat, m1, m2 = _c1(p0, root0, b0.reshape(1, HID), lw, lb)

    gpad = jnp.pad(bn_g.reshape(N_GROUPS, HID), ((0, G_PAD - N_GROUPS), (0, 0)))
    bpad = jnp.pad(bn_b.reshape(N_GROUPS, HID), ((0, G_PAD - N_GROUPS), (0, 0)))
    ylo, yhi, hroot = _c2(out0, smat, m1, m2, gpad, bpad, W1, root1,
                          b1.reshape(1, HID))

    p1 = _edge_pass(ylo.reshape(NR, HALF), yhi.reshape(NR, HALF), meta, inv)
    p1 = p1.reshape(NC, N_NODES, HALF)

    return _fin(p1, hroot)
